# Initial kernel scaffold; baseline (speedup 1.0000x reference)
import jax
import jax.numpy as jnp
from jax.experimental import pallas as pl


def kernel(node_features, rigid_rots, rigid_trans, edge_features, res_mask, noising_mask, W_pts, b_pts, W_m1, b_m1, W_m2, b_m2, W_n1, b_n1, W_n2, b_n2, g_s, b_s, W_e1, b_e1, W_e2, b_e2, g_z, b_z, W_bb, b_bb, edge_index):
    raise NotImplementedError("write your pallas kernel here")


# speedup vs baseline: 1.1701x; 1.1701x over previous
import functools

import jax
import jax.numpy as jnp
import numpy as np
from jax import lax
from jax.experimental import pallas as pl
from jax.experimental.pallas import tpu as pltpu
from jax.experimental.pallas import tpu_sc as plsc

_N = 10000
_E = 160000
_NP = 10240
_EP = 163840

_F32 = jnp.float32


def _nodeprep_body(nf_ref, wpts_ref, bpts_ref, wa_ref, wb_ref,
                   a_ref, b_ref, pts_ref):
    nf = nf_ref[...]
    a_ref[...] = jnp.dot(nf, wa_ref[...], preferred_element_type=_F32)
    b_ref[...] = jnp.dot(nf, wb_ref[...], preferred_element_type=_F32)
    pts_ref[...] = jnp.dot(nf, wpts_ref[...], preferred_element_type=_F32) + bpts_ref[...]


def _gpt_body(pts_ref, rot_ref, tr_ref, out_ref):
    pts = pts_ref[...]
    rot = rot_ref[...]
    tr = tr_ref[...]
    rows = []
    for p in range(8):
        for i in range(3):
            acc = tr[i:i + 1, :]
            for j in range(3):
                acc = acc + rot[3 * i + j:3 * i + j + 1, :] * pts[3 * p + j:3 * p + j + 1, :]
            rows.append(acc)
    zero = jnp.zeros_like(rows[0])
    out_ref[...] = jnp.concatenate(rows + [zero] * 8, axis=0)


def _msg_body(gs_ref, gd_ref, ef_ref, sel_ref, wd_ref, wef_ref, bm1_ref, r1_ref):
    gs = gs_ref[...]
    gd = gd_ref[...]
    dg = gs[:, 128:160] - gd[:, 128:160]
    d2 = jnp.dot(dg * dg, sel_ref[...], preferred_element_type=_F32)
    d = jnp.sqrt(d2 + 1e-8)
    h = gs[:, :128] + gd[:, :128]
    h = h + jnp.dot(ef_ref[...], wef_ref[...], preferred_element_type=_F32)
    h = h + jnp.dot(d, wd_ref[...], preferred_element_type=_F32)
    r1_ref[...] = jnp.maximum(h + bm1_ref[...], 0.0)


def _node_body(nf_ref, aggh_ref, deg_ref, wm2_ref, bm2_ref, wn1a_ref, wn1b_ref,
               bn1_ref, wn2_ref, bn2_ref, gsn_ref, bsn_ref, rm_ref, nm_ref,
               we1s_ref, we1d_ref, wbb_ref, bbb_ref,
               s_ref, as_ref, bsd_ref, upd_ref):
    nf = nf_ref[...]
    deg = deg_ref[:, 0:1]
    agg = jnp.dot(aggh_ref[...], wm2_ref[...], preferred_element_type=_F32)
    agg = agg / jnp.maximum(deg, 1.0) + bm2_ref[...] * jnp.minimum(deg, 1.0)
    h = jnp.dot(nf, wn1a_ref[...], preferred_element_type=_F32)
    h = h + jnp.dot(agg, wn1b_ref[...], preferred_element_type=_F32)
    h = jnp.maximum(h + bn1_ref[...], 0.0)
    su = nf + jnp.dot(h, wn2_ref[...], preferred_element_type=_F32) + bn2_ref[...]
    mu = jnp.mean(su, axis=1, keepdims=True)
    var = jnp.mean((su - mu) * (su - mu), axis=1, keepdims=True)
    s = (su - mu) / jnp.sqrt(var + 1e-5) * gsn_ref[...] + bsn_ref[...]
    s = s * rm_ref[...]
    s_ref[...] = s
    as_ref[...] = jnp.dot(s, we1s_ref[...], preferred_element_type=_F32)
    bsd_ref[...] = jnp.dot(s, we1d_ref[...], preferred_element_type=_F32)
    sm = s * nm_ref[...]
    upd_ref[...] = jnp.dot(sm, wbb_ref[...], preferred_element_type=_F32) * nm_ref[...]


def _bb_body(upd_ref, rot_ref, tr_ref, nrot_ref, ntr_ref):
    u = upd_ref[...]
    b = u[0:1]; c = u[1:2]; d = u[2:3]
    inv = lax.rsqrt(1.0 + b * b + c * c + d * d)
    w = inv; x = b * inv; y = c * inv; z = d * inv
    ru = [1 - 2 * (y * y + z * z), 2 * (x * y - w * z), 2 * (x * z + w * y),
          2 * (x * y + w * z), 1 - 2 * (x * x + z * z), 2 * (y * z - w * x),
          2 * (x * z - w * y), 2 * (y * z + w * x), 1 - 2 * (x * x + y * y)]
    rot = rot_ref[...]
    out_rows = []
    for i in range(3):
        for k in range(3):
            acc = rot[3 * i + 0:3 * i + 1] * ru[0 * 3 + k]
            acc = acc + rot[3 * i + 1:3 * i + 2] * ru[1 * 3 + k]
            acc = acc + rot[3 * i + 2:3 * i + 3] * ru[2 * 3 + k]
            out_rows.append(acc)
    zero = jnp.zeros_like(out_rows[0])
    nrot_ref[...] = jnp.concatenate(out_rows + [zero] * 7, axis=0)
    tr = tr_ref[...]
    trows = []
    for i in range(3):
        acc = tr[i:i + 1]
        for j in range(3):
            acc = acc + rot[3 * i + j:3 * i + j + 1] * u[3 + j:4 + j]
        trows.append(acc)
    ntr_ref[...] = jnp.concatenate(trows + [zero] * 5, axis=0)


def _edge_body(ef_ref, gs_ref, gd_ref, we1e_ref, be1_ref, we2_ref, be2_ref,
               gz_ref, bz_ref, z_ref):
    ef = ef_ref[...]
    h = jnp.dot(ef, we1e_ref[...], preferred_element_type=_F32)
    h = jnp.maximum(h + gs_ref[...] + gd_ref[...] + be1_ref[...], 0.0)
    zz = ef + jnp.dot(h, we2_ref[...], preferred_element_type=_F32) + be2_ref[...]
    mu = jnp.mean(zz, axis=1, keepdims=True)
    var = jnp.mean((zz - mu) * (zz - mu), axis=1, keepdims=True)
    z_ref[...] = (zz - mu) / jnp.sqrt(var + 1e-5) * gz_ref[...] + bz_ref[...]


def _full(r, c):
    return pl.BlockSpec((r, c), lambda i: (0, 0))


def _rows(b, c):
    return pl.BlockSpec((b, c), lambda i: (i, 0))


_SEL = np.zeros((32, 8), dtype=np.float32)
for _p in range(8):
    for _j in range(3):
        _SEL[3 * _p + _j, _p] = 1.0


def kernel(node_features, rigid_rots, rigid_trans, edge_features, res_mask,
           noising_mask, W_pts, b_pts, W_m1, b_m1, W_m2, b_m2, W_n1, b_n1,
           W_n2, b_n2, g_s, b_s, W_e1, b_e1, W_e2, b_e2, g_z, b_z, W_bb,
           b_bb, edge_index):
    src_p = jnp.concatenate([edge_index[0], jnp.zeros((_EP - _E,), jnp.int32)])
    dst_p = jnp.concatenate([edge_index[1], jnp.full((_EP - _E,), _NP - 1, jnp.int32)])
    nf = jnp.pad(node_features, ((0, _NP - _N), (0, 0)))
    ef = jnp.pad(edge_features, ((0, _EP - _E), (0, 0)))
    rotsT = jnp.pad(rigid_rots.reshape(_N, 9), ((0, _NP - _N), (0, 7))).T
    trT = jnp.pad(rigid_trans, ((0, _NP - _N), (0, 5))).T
    rm = jnp.pad(res_mask, (0, _NP - _N)).reshape(_NP, 1)
    nm = jnp.pad(noising_mask, (0, _NP - _N)).reshape(_NP, 1)
    wptsP = jnp.pad(W_pts, ((0, 0), (0, 8)))
    bptsP = jnp.pad(b_pts, (0, 8)).reshape(1, 32)
    wm1a, wm1b, wef1, wd1 = W_m1[:256], W_m1[256:512], W_m1[512:640], W_m1[640:648]
    we1e, we1s, we1d = W_e1[:128], W_e1[128:384], W_e1[384:640]
    wn1a, wn1b = W_n1[:256], W_n1[256:512]
    wbbP = jnp.pad(W_bb, ((0, 0), (0, 2)))
    bbbP = jnp.pad(b_bb, (0, 2)).reshape(1, 8)
    sel = jnp.asarray(_SEL)
    r2 = lambda v: v.reshape(1, -1)

    a_tab, b_tab, pts = pl.pallas_call(
        _nodeprep_body,
        grid=(_NP // 1024,),
        in_specs=[_rows(1024, 256), _full(256, 32), _full(1, 32),
                  _full(256, 128), _full(256, 128)],
        out_specs=[_rows(1024, 128), _rows(1024, 128), _rows(1024, 32)],
        out_shape=[jax.ShapeDtypeStruct((_NP, 128), _F32),
                   jax.ShapeDtypeStruct((_NP, 128), _F32),
                   jax.ShapeDtypeStruct((_NP, 32), _F32)],
    )(nf, wptsP, bptsP, wm1a, wm1b)

    gpT = pl.pallas_call(
        _gpt_body,
        grid=(_NP // 2048,),
        in_specs=[_rows_t(32, 2048), _rows_t(16, 2048), _rows_t(8, 2048)],
        out_specs=_rows_t(32, 2048),
        out_shape=jax.ShapeDtypeStruct((32, _NP), _F32),
    )(pts.T, rotsT, trT)
    gp_nodes = gpT.T
    t_src = jnp.concatenate([a_tab, gp_nodes], axis=1)
    t_dst = jnp.concatenate([b_tab, gp_nodes], axis=1)

    gs1 = jnp.take(t_src, src_p, axis=0)
    gd1 = jnp.take(t_dst, dst_p, axis=0)

    r1 = pl.pallas_call(
        _msg_body,
        grid=(_EP // 2048,),
        in_specs=[_rows(2048, 160), _rows(2048, 160), _rows(2048, 128),
                  _full(32, 8), _full(8, 128), _full(128, 128), _full(1, 128)],
        out_specs=_rows(2048, 128),
        out_shape=jax.ShapeDtypeStruct((_EP, 128), _F32),
    )(gs1, gd1, ef, sel, wd1, wef1, r2(b_m1))

    aggh = jax.ops.segment_sum(r1, dst_p, num_segments=_NP)
    deg = jax.ops.segment_sum(jnp.ones((_EP,), _F32), dst_p, num_segments=_NP)
    deg16 = jnp.broadcast_to(deg[:, None], (_NP, 16))

    s, as_tab, bs_tab, upd = pl.pallas_call(
        _node_body,
        grid=(_NP // 1024,),
        in_specs=[_rows(1024, 256), _rows(1024, 128), _rows(1024, 16),
                  _full(128, 256), _full(1, 256), _full(256, 128), _full(256, 128),
                  _full(1, 128), _full(128, 256), _full(1, 256), _full(1, 256),
                  _full(1, 256), _rows(1024, 1), _rows(1024, 1),
                  _full(256, 128), _full(256, 128), _full(256, 8), _full(1, 8)],
        out_specs=[_rows(1024, 256), _rows(1024, 128), _rows(1024, 128),
                   _rows(1024, 8)],
        out_shape=[jax.ShapeDtypeStruct((_NP, 256), _F32),
                   jax.ShapeDtypeStruct((_NP, 128), _F32),
                   jax.ShapeDtypeStruct((_NP, 128), _F32),
                   jax.ShapeDtypeStruct((_NP, 8), _F32)],
    )(nf, aggh, deg16, W_m2, r2(b_m2), wn1a, wn1b, r2(b_n1), W_n2, r2(b_n2),
      r2(g_s), r2(b_s), rm, nm, we1s, we1d, wbbP, bbbP)

    nrotT, ntrT = pl.pallas_call(
        _bb_body,
        grid=(_NP // 2048,),
        in_specs=[_rows_t(8, 2048), _rows_t(16, 2048), _rows_t(8, 2048)],
        out_specs=[_rows_t(16, 2048), _rows_t(8, 2048)],
        out_shape=[jax.ShapeDtypeStruct((16, _NP), _F32),
                   jax.ShapeDtypeStruct((8, _NP), _F32)],
    )(upd.T, rotsT, trT)

    gs2 = jnp.take(as_tab, src_p, axis=0)
    gd2 = jnp.take(bs_tab, dst_p, axis=0)

    z = pl.pallas_call(
        _edge_body,
        grid=(_EP // 2048,),
        in_specs=[_rows(2048, 128), _rows(2048, 128), _rows(2048, 128),
                  _full(128, 128), _full(1, 128), _full(128, 128), _full(1, 128),
                  _full(1, 128), _full(1, 128)],
        out_specs=_rows(2048, 128),
        out_shape=jax.ShapeDtypeStruct((_EP, 128), _F32),
    )(ef, gs2, gd2, we1e, r2(b_e1), W_e2, r2(b_e2), r2(g_z), r2(b_z))

    s_out = s[:_N]
    new_rots = nrotT.T[:_N, :9].reshape(_N, 3, 3)
    new_trans = ntrT.T[:_N, :3]
    return (s_out, new_rots, new_trans, z[:_E])


def _rows_t(r, b):
    return pl.BlockSpec((r, b), lambda i: (0, i))


# speedup vs baseline: 4.2212x; 3.6075x over previous
import functools

import jax
import jax.numpy as jnp
import numpy as np
from jax import lax
from jax.experimental import pallas as pl
from jax.experimental.pallas import tpu as pltpu
from jax.experimental.pallas import tpu_sc as plsc

_N = 10000
_E = 160000
_NP = 10240
_EP = 163840

_F32 = jnp.float32


def _nodeprep_body(nf_ref, wpts_ref, bpts_ref, wa_ref, wb_ref,
                   a_ref, b_ref, pts_ref):
    nf = nf_ref[...]
    a_ref[...] = jnp.dot(nf, wa_ref[...], preferred_element_type=_F32)
    b_ref[...] = jnp.dot(nf, wb_ref[...], preferred_element_type=_F32)
    pts_ref[...] = jnp.dot(nf, wpts_ref[...], preferred_element_type=_F32) + bpts_ref[...]


def _gpt_body(pts_ref, rot_ref, tr_ref, out_ref):
    pts = pts_ref[...]
    rot = rot_ref[...]
    tr = tr_ref[...]
    rows = []
    for p in range(8):
        for i in range(3):
            acc = tr[i:i + 1, :]
            for j in range(3):
                acc = acc + rot[3 * i + j:3 * i + j + 1, :] * pts[3 * p + j:3 * p + j + 1, :]
            rows.append(acc)
    zero = jnp.zeros_like(rows[0])
    out_ref[...] = jnp.concatenate(rows + [zero] * 8, axis=0)


def _msg_body(gs_ref, gd_ref, ef_ref, sel_ref, wd_ref, wef_ref, bm1_ref, r1_ref):
    gs = gs_ref[...]
    gd = gd_ref[...]
    dg = gs[:, 128:160] - gd[:, 128:160]
    d2 = jnp.dot(dg * dg, sel_ref[...], preferred_element_type=_F32)
    d = jnp.sqrt(d2 + 1e-8)
    h = gs[:, :128] + gd[:, :128]
    h = h + jnp.dot(ef_ref[...], wef_ref[...], preferred_element_type=_F32)
    h = h + jnp.dot(d, wd_ref[...], preferred_element_type=_F32)
    r1_ref[...] = jnp.maximum(h + bm1_ref[...], 0.0)


def _node_body(nf_ref, aggh_ref, deg_ref, wm2_ref, bm2_ref, wn1a_ref, wn1b_ref,
               bn1_ref, wn2_ref, bn2_ref, gsn_ref, bsn_ref, rm_ref, nm_ref,
               we1s_ref, we1d_ref, wbb_ref, bbb_ref,
               s_ref, as_ref, bsd_ref, upd_ref):
    nf = nf_ref[...]
    deg = deg_ref[:, 0:1]
    agg = jnp.dot(aggh_ref[...], wm2_ref[...], preferred_element_type=_F32)
    agg = agg / jnp.maximum(deg, 1.0) + bm2_ref[...] * jnp.minimum(deg, 1.0)
    h = jnp.dot(nf, wn1a_ref[...], preferred_element_type=_F32)
    h = h + jnp.dot(agg, wn1b_ref[...], preferred_element_type=_F32)
    h = jnp.maximum(h + bn1_ref[...], 0.0)
    su = nf + jnp.dot(h, wn2_ref[...], preferred_element_type=_F32) + bn2_ref[...]
    mu = jnp.mean(su, axis=1, keepdims=True)
    var = jnp.mean((su - mu) * (su - mu), axis=1, keepdims=True)
    s = (su - mu) / jnp.sqrt(var + 1e-5) * gsn_ref[...] + bsn_ref[...]
    s = s * rm_ref[...]
    s_ref[...] = s
    as_ref[...] = jnp.dot(s, we1s_ref[...], preferred_element_type=_F32)
    bsd_ref[...] = jnp.dot(s, we1d_ref[...], preferred_element_type=_F32)
    sm = s * nm_ref[...]
    upd_ref[...] = jnp.dot(sm, wbb_ref[...], preferred_element_type=_F32) * nm_ref[...]


def _bb_body(upd_ref, rot_ref, tr_ref, nrot_ref, ntr_ref):
    u = upd_ref[...]
    b = u[0:1]; c = u[1:2]; d = u[2:3]
    inv = lax.rsqrt(1.0 + b * b + c * c + d * d)
    w = inv; x = b * inv; y = c * inv; z = d * inv
    ru = [1 - 2 * (y * y + z * z), 2 * (x * y - w * z), 2 * (x * z + w * y),
          2 * (x * y + w * z), 1 - 2 * (x * x + z * z), 2 * (y * z - w * x),
          2 * (x * z - w * y), 2 * (y * z + w * x), 1 - 2 * (x * x + y * y)]
    rot = rot_ref[...]
    out_rows = []
    for i in range(3):
        for k in range(3):
            acc = rot[3 * i + 0:3 * i + 1] * ru[0 * 3 + k]
            acc = acc + rot[3 * i + 1:3 * i + 2] * ru[1 * 3 + k]
            acc = acc + rot[3 * i + 2:3 * i + 3] * ru[2 * 3 + k]
            out_rows.append(acc)
    zero = jnp.zeros_like(out_rows[0])
    nrot_ref[...] = jnp.concatenate(out_rows + [zero] * 7, axis=0)
    tr = tr_ref[...]
    trows = []
    for i in range(3):
        acc = tr[i:i + 1]
        for j in range(3):
            acc = acc + rot[3 * i + j:3 * i + j + 1] * u[3 + j:4 + j]
        trows.append(acc)
    ntr_ref[...] = jnp.concatenate(trows + [zero] * 5, axis=0)


def _edge_body(ef_ref, gs_ref, gd_ref, we1e_ref, be1_ref, we2_ref, be2_ref,
               gz_ref, bz_ref, z_ref):
    ef = ef_ref[...]
    h = jnp.dot(ef, we1e_ref[...], preferred_element_type=_F32)
    h = jnp.maximum(h + gs_ref[...] + gd_ref[...] + be1_ref[...], 0.0)
    zz = ef + jnp.dot(h, we2_ref[...], preferred_element_type=_F32) + be2_ref[...]
    mu = jnp.mean(zz, axis=1, keepdims=True)
    var = jnp.mean((zz - mu) * (zz - mu), axis=1, keepdims=True)
    z_ref[...] = (zz - mu) / jnp.sqrt(var + 1e-5) * gz_ref[...] + bz_ref[...]


_NW = 32
_PERW = _EP // _NW
_NJ = _PERW // 128
_NPS = _NP // 16


def _sc_gather_pair(d_width):
    mesh = plsc.VectorSubcoreMesh(core_axis_name="c", subcore_axis_name="s")

    @functools.partial(
        pl.kernel,
        mesh=mesh,
        out_type=[jax.ShapeDtypeStruct((_EP, d_width), _F32),
                  jax.ShapeDtypeStruct((_EP, d_width), _F32)],
        scratch_types=[
            pltpu.VMEM((_NJ, 128), jnp.int32),
            pltpu.VMEM((_NJ, 128), jnp.int32),
            pltpu.VMEM((128, d_width), _F32),
            pltpu.VMEM((128, d_width), _F32),
            pltpu.SemaphoreType.DMA,
            pltpu.SemaphoreType.DMA,
        ],
    )
    def gather(tab_s, tab_d, idx_s, idx_d, out_s, out_d,
               iv_s, iv_d, buf_s, buf_d, s_sem, d_sem):
        wid = lax.axis_index("s") * 2 + lax.axis_index("c")
        base = wid * _PERW
        pltpu.sync_copy(idx_s.at[wid], iv_s)
        pltpu.sync_copy(idx_d.at[wid], iv_d)

        def body(j, carry):
            off = pl.multiple_of(base + j * 128, 128)
            pltpu.async_copy(tab_s.at[iv_s.at[j]], buf_s, s_sem).wait()
            pltpu.async_copy(tab_d.at[iv_d.at[j]], buf_d, d_sem).wait()
            pltpu.async_copy(buf_s, out_s.at[pl.ds(off, 128)], s_sem).wait()
            pltpu.async_copy(buf_d, out_d.at[pl.ds(off, 128)], d_sem).wait()
            return carry

        lax.fori_loop(0, _NJ, body, 0)

    return gather


def _sc_scatter():
    mesh = plsc.VectorSubcoreMesh(core_axis_name="c", subcore_axis_name="s")

    @functools.partial(
        pl.kernel,
        mesh=mesh,
        out_type=jax.ShapeDtypeStruct((2 * _NP, 128), _F32),
        scratch_types=[
            pltpu.VMEM((64,), jnp.int32),
            pltpu.VMEM((64, 128), _F32),
            pltpu.VMEM_SHARED((_NP, 128), _F32),
        ],
    )
    def scat(r1_hbm, dsti_hbm, z128_hbm, aggh_out, ivc, rowbuf, acc):
        cid = lax.axis_index("c")
        sid = lax.axis_index("s")
        wid = sid * 2 + cid
        base = wid * _PERW
        pltpu.sync_copy(z128_hbm, rowbuf)
        for k in range(_NPS // 64):
            ns = pl.ds(sid * _NPS + k * 64, 64)
            pltpu.sync_copy(rowbuf, acc.at[ns])
        plsc.subcore_barrier()

        def body(j, carry):
            off = pl.multiple_of(base + j * 64, 64)
            pltpu.sync_copy(dsti_hbm.at[wid, j], ivc)
            pltpu.sync_copy(r1_hbm.at[pl.ds(off, 64)], rowbuf)
            pltpu.sync_copy(rowbuf, acc.at[ivc], add=True)
            return carry

        lax.fori_loop(0, 2 * _NJ, body, 0)
        plsc.subcore_barrier()
        for k in range(_NPS // 64):
            ns = pl.ds(sid * _NPS + k * 64, 64)
            oo = pl.ds(cid * _NP + sid * _NPS + k * 64, 64)
            pltpu.sync_copy(acc.at[ns], rowbuf)
            pltpu.sync_copy(rowbuf, aggh_out.at[oo])

    return scat


def _sc_degree():
    mesh = plsc.VectorSubcoreMesh(core_axis_name="c", subcore_axis_name="s")

    @functools.partial(
        pl.kernel,
        mesh=mesh,
        out_type=jax.ShapeDtypeStruct((2 * _NP, 128), _F32),
        scratch_types=[
            pltpu.VMEM((64,), jnp.int32),
            pltpu.VMEM((64, 128), _F32),
            pltpu.VMEM_SHARED((_NP, 128), _F32),
        ],
    )
    def degk(dsti_hbm, z128_hbm, ones_hbm, deg_out, ivc, buf, dacc):
        cid = lax.axis_index("c")
        sid = lax.axis_index("s")
        wid = sid * 2 + cid
        pltpu.sync_copy(z128_hbm, buf)
        for k in range(_NPS // 64):
            ns = pl.ds(sid * _NPS + k * 64, 64)
            pltpu.sync_copy(buf, dacc.at[ns])
        pltpu.sync_copy(ones_hbm, buf)
        plsc.subcore_barrier()

        def body(j, carry):
            pltpu.sync_copy(dsti_hbm.at[wid, j], ivc)
            pltpu.sync_copy(buf, dacc.at[ivc], add=True)
            return carry

        lax.fori_loop(0, 2 * _NJ, body, 0)
        plsc.subcore_barrier()
        for k in range(_NPS // 64):
            ns = pl.ds(sid * _NPS + k * 64, 64)
            oo = pl.ds(cid * _NP + sid * _NPS + k * 64, 64)
            pltpu.sync_copy(dacc.at[ns], buf)
            pltpu.sync_copy(buf, deg_out.at[oo])

    return degk


def _full(r, c):
    return pl.BlockSpec((r, c), lambda i: (0, 0))


def _rows(b, c):
    return pl.BlockSpec((b, c), lambda i: (i, 0))


_SEL = np.zeros((32, 8), dtype=np.float32)
for _p in range(8):
    for _j in range(3):
        _SEL[3 * _p + _j, _p] = 1.0


def kernel(node_features, rigid_rots, rigid_trans, edge_features, res_mask,
           noising_mask, W_pts, b_pts, W_m1, b_m1, W_m2, b_m2, W_n1, b_n1,
           W_n2, b_n2, g_s, b_s, W_e1, b_e1, W_e2, b_e2, g_z, b_z, W_bb,
           b_bb, edge_index):
    pad_idx = _N + (jnp.arange(_EP - _E, dtype=jnp.int32) % (_NP - _N))
    src_p = jnp.concatenate([edge_index[0], pad_idx])
    dst_p = jnp.concatenate([edge_index[1], pad_idx])
    nf = jnp.pad(node_features, ((0, _NP - _N), (0, 0)))
    ef = jnp.pad(edge_features, ((0, _EP - _E), (0, 0)))
    rotsT = jnp.pad(rigid_rots.reshape(_N, 9), ((0, _NP - _N), (0, 7))).T
    trT = jnp.pad(rigid_trans, ((0, _NP - _N), (0, 5))).T
    rm = jnp.pad(res_mask, (0, _NP - _N)).reshape(_NP, 1)
    nm = jnp.pad(noising_mask, (0, _NP - _N)).reshape(_NP, 1)
    wptsP = jnp.pad(W_pts, ((0, 0), (0, 8)))
    bptsP = jnp.pad(b_pts, (0, 8)).reshape(1, 32)
    wm1a, wm1b, wef1, wd1 = W_m1[:256], W_m1[256:512], W_m1[512:640], W_m1[640:648]
    we1e, we1s, we1d = W_e1[:128], W_e1[128:384], W_e1[384:640]
    wn1a, wn1b = W_n1[:256], W_n1[256:512]
    wbbP = jnp.pad(W_bb, ((0, 0), (0, 2)))
    bbbP = jnp.pad(b_bb, (0, 2)).reshape(1, 8)
    sel = jnp.asarray(_SEL)
    r2 = lambda v: v.reshape(1, -1)

    a_tab, b_tab, pts = pl.pallas_call(
        _nodeprep_body,
        grid=(_NP // 1024,),
        in_specs=[_rows(1024, 256), _full(256, 32), _full(1, 32),
                  _full(256, 128), _full(256, 128)],
        out_specs=[_rows(1024, 128), _rows(1024, 128), _rows(1024, 32)],
        out_shape=[jax.ShapeDtypeStruct((_NP, 128), _F32),
                   jax.ShapeDtypeStruct((_NP, 128), _F32),
                   jax.ShapeDtypeStruct((_NP, 32), _F32)],
    )(nf, wptsP, bptsP, wm1a, wm1b)

    gpT = pl.pallas_call(
        _gpt_body,
        grid=(_NP // 2048,),
        in_specs=[_rows_t(32, 2048), _rows_t(16, 2048), _rows_t(8, 2048)],
        out_specs=_rows_t(32, 2048),
        out_shape=jax.ShapeDtypeStruct((32, _NP), _F32),
    )(pts.T, rotsT, trT)
    gp_nodes = gpT.T
    padc = jnp.zeros((_NP, 96), _F32)
    t_src = jnp.concatenate([a_tab, gp_nodes, padc], axis=1)
    t_dst = jnp.concatenate([b_tab, gp_nodes, padc], axis=1)

    src3 = src_p.reshape(_NW, _NJ, 128)
    dst3 = dst_p.reshape(_NW, _NJ, 128)
    gs1, gd1 = _sc_gather_pair(256)(t_src, t_dst, src3, dst3)

    r1 = pl.pallas_call(
        _msg_body,
        grid=(_EP // 2048,),
        in_specs=[_rows(2048, 256), _rows(2048, 256), _rows(2048, 128),
                  _full(32, 8), _full(8, 128), _full(128, 128), _full(1, 128)],
        out_specs=_rows(2048, 128),
        out_shape=jax.ShapeDtypeStruct((_EP, 128), _F32),
    )(gs1, gd1, ef, sel, wd1, wef1, r2(b_m1))

    dst3s = dst_p.reshape(_NW, 2 * _NJ, 64)
    z64 = jnp.zeros((64, 128), _F32)
    agg2 = _sc_scatter()(r1, dst3s, z64)
    degf = _sc_degree()(dst3s, z64, jnp.ones((64, 128), _F32))
    aggh = agg2[:_NP] + agg2[_NP:]
    deg16 = (degf[:_NP] + degf[_NP:])[:, :16]

    s, as_tab, bs_tab, upd = pl.pallas_call(
        _node_body,
        grid=(_NP // 1024,),
        in_specs=[_rows(1024, 256), _rows(1024, 128), _rows(1024, 16),
                  _full(128, 256), _full(1, 256), _full(256, 128), _full(256, 128),
                  _full(1, 128), _full(128, 256), _full(1, 256), _full(1, 256),
                  _full(1, 256), _rows(1024, 1), _rows(1024, 1),
                  _full(256, 128), _full(256, 128), _full(256, 8), _full(1, 8)],
        out_specs=[_rows(1024, 256), _rows(1024, 128), _rows(1024, 128),
                   _rows(1024, 8)],
        out_shape=[jax.ShapeDtypeStruct((_NP, 256), _F32),
                   jax.ShapeDtypeStruct((_NP, 128), _F32),
                   jax.ShapeDtypeStruct((_NP, 128), _F32),
                   jax.ShapeDtypeStruct((_NP, 8), _F32)],
    )(nf, aggh, deg16, W_m2, r2(b_m2), wn1a, wn1b, r2(b_n1), W_n2, r2(b_n2),
      r2(g_s), r2(b_s), rm, nm, we1s, we1d, wbbP, bbbP)

    nrotT, ntrT = pl.pallas_call(
        _bb_body,
        grid=(_NP // 2048,),
        in_specs=[_rows_t(8, 2048), _rows_t(16, 2048), _rows_t(8, 2048)],
        out_specs=[_rows_t(16, 2048), _rows_t(8, 2048)],
        out_shape=[jax.ShapeDtypeStruct((16, _NP), _F32),
                   jax.ShapeDtypeStruct((8, _NP), _F32)],
    )(upd.T, rotsT, trT)

    gs2, gd2 = _sc_gather_pair(128)(as_tab, bs_tab, src3, dst3)

    z = pl.pallas_call(
        _edge_body,
        grid=(_EP // 2048,),
        in_specs=[_rows(2048, 128), _rows(2048, 128), _rows(2048, 128),
                  _full(128, 128), _full(1, 128), _full(128, 128), _full(1, 128),
                  _full(1, 128), _full(1, 128)],
        out_specs=_rows(2048, 128),
        out_shape=jax.ShapeDtypeStruct((_EP, 128), _F32),
    )(ef, gs2, gd2, we1e, r2(b_e1), W_e2, r2(b_e2), r2(g_z), r2(b_z))

    s_out = s[:_N]
    new_rots = nrotT.T[:_N, :9].reshape(_N, 3, 3)
    new_trans = ntrT.T[:_N, :3]
    return (s_out, new_rots, new_trans, z[:_E])


def _rows_t(r, b):
    return pl.BlockSpec((r, b), lambda i: (0, i))


# speedup vs baseline: 4.5642x; 1.0812x over previous
import functools

import jax
import jax.numpy as jnp
import numpy as np
from jax import lax
from jax.experimental import pallas as pl
from jax.experimental.pallas import tpu as pltpu
from jax.experimental.pallas import tpu_sc as plsc

_N = 10000
_E = 160000
_NP = 10240
_EP = 163840

_F32 = jnp.float32


def _nodeprep_body(nf_ref, wpts_ref, bpts_ref, wa_ref, wb_ref,
                   a_ref, b_ref, pts_ref):
    nf = nf_ref[...]
    a_ref[...] = jnp.dot(nf, wa_ref[...], preferred_element_type=_F32)
    b_ref[...] = jnp.dot(nf, wb_ref[...], preferred_element_type=_F32)
    pts_ref[...] = jnp.dot(nf, wpts_ref[...], preferred_element_type=_F32) + bpts_ref[...]


def _gpt_body(pts_ref, rot_ref, tr_ref, out_ref):
    pts = pts_ref[...]
    rot = rot_ref[...]
    tr = tr_ref[...]
    rows = []
    for p in range(8):
        for i in range(3):
            acc = tr[i:i + 1, :]
            for j in range(3):
                acc = acc + rot[3 * i + j:3 * i + j + 1, :] * pts[3 * p + j:3 * p + j + 1, :]
            rows.append(acc)
    zero = jnp.zeros_like(rows[0])
    out_ref[...] = jnp.concatenate(rows + [zero] * 8, axis=0)


def _msg_body(gs_ref, gd_ref, ef_ref, sel_ref, wd_ref, wef_ref, bm1_ref, r1_ref):
    gs = gs_ref[...]
    gd = gd_ref[...]
    dg = gs[:, 128:160] - gd[:, 128:160]
    d2 = jnp.dot(dg * dg, sel_ref[...], preferred_element_type=_F32)
    d = jnp.sqrt(d2 + 1e-8)
    h = gs[:, :128] + gd[:, :128]
    h = h + jnp.dot(ef_ref[...], wef_ref[...], preferred_element_type=_F32)
    h = h + jnp.dot(d, wd_ref[...], preferred_element_type=_F32)
    r1_ref[...] = jnp.maximum(h + bm1_ref[...], 0.0)


def _node_body(nf_ref, aggh_ref, deg_ref, wm2_ref, bm2_ref, wn1a_ref, wn1b_ref,
               bn1_ref, wn2_ref, bn2_ref, gsn_ref, bsn_ref, rm_ref, nm_ref,
               we1s_ref, we1d_ref, wbb_ref, bbb_ref,
               s_ref, as_ref, bsd_ref, upd_ref):
    nf = nf_ref[...]
    deg = deg_ref[:, 0:1]
    agg = jnp.dot(aggh_ref[...], wm2_ref[...], preferred_element_type=_F32)
    agg = agg / jnp.maximum(deg, 1.0) + bm2_ref[...] * jnp.minimum(deg, 1.0)
    h = jnp.dot(nf, wn1a_ref[...], preferred_element_type=_F32)
    h = h + jnp.dot(agg, wn1b_ref[...], preferred_element_type=_F32)
    h = jnp.maximum(h + bn1_ref[...], 0.0)
    su = nf + jnp.dot(h, wn2_ref[...], preferred_element_type=_F32) + bn2_ref[...]
    mu = jnp.mean(su, axis=1, keepdims=True)
    var = jnp.mean((su - mu) * (su - mu), axis=1, keepdims=True)
    s = (su - mu) / jnp.sqrt(var + 1e-5) * gsn_ref[...] + bsn_ref[...]
    s = s * rm_ref[...]
    s_ref[...] = s
    as_ref[...] = jnp.dot(s, we1s_ref[...], preferred_element_type=_F32)
    bsd_ref[...] = jnp.dot(s, we1d_ref[...], preferred_element_type=_F32)
    sm = s * nm_ref[...]
    upd_ref[...] = jnp.dot(sm, wbb_ref[...], preferred_element_type=_F32) * nm_ref[...]


def _bb_body(upd_ref, rot_ref, tr_ref, nrot_ref, ntr_ref):
    u = upd_ref[...]
    b = u[0:1]; c = u[1:2]; d = u[2:3]
    inv = lax.rsqrt(1.0 + b * b + c * c + d * d)
    w = inv; x = b * inv; y = c * inv; z = d * inv
    ru = [1 - 2 * (y * y + z * z), 2 * (x * y - w * z), 2 * (x * z + w * y),
          2 * (x * y + w * z), 1 - 2 * (x * x + z * z), 2 * (y * z - w * x),
          2 * (x * z - w * y), 2 * (y * z + w * x), 1 - 2 * (x * x + y * y)]
    rot = rot_ref[...]
    out_rows = []
    for i in range(3):
        for k in range(3):
            acc = rot[3 * i + 0:3 * i + 1] * ru[0 * 3 + k]
            acc = acc + rot[3 * i + 1:3 * i + 2] * ru[1 * 3 + k]
            acc = acc + rot[3 * i + 2:3 * i + 3] * ru[2 * 3 + k]
            out_rows.append(acc)
    zero = jnp.zeros_like(out_rows[0])
    nrot_ref[...] = jnp.concatenate(out_rows + [zero] * 7, axis=0)
    tr = tr_ref[...]
    trows = []
    for i in range(3):
        acc = tr[i:i + 1]
        for j in range(3):
            acc = acc + rot[3 * i + j:3 * i + j + 1] * u[3 + j:4 + j]
        trows.append(acc)
    ntr_ref[...] = jnp.concatenate(trows + [zero] * 5, axis=0)


def _edge_body(ef_ref, gs_ref, gd_ref, we1e_ref, be1_ref, we2_ref, be2_ref,
               gz_ref, bz_ref, z_ref):
    ef = ef_ref[...]
    h = jnp.dot(ef, we1e_ref[...], preferred_element_type=_F32)
    h = jnp.maximum(h + gs_ref[...] + gd_ref[...] + be1_ref[...], 0.0)
    zz = ef + jnp.dot(h, we2_ref[...], preferred_element_type=_F32) + be2_ref[...]
    mu = jnp.mean(zz, axis=1, keepdims=True)
    var = jnp.mean((zz - mu) * (zz - mu), axis=1, keepdims=True)
    z_ref[...] = (zz - mu) / jnp.sqrt(var + 1e-5) * gz_ref[...] + bz_ref[...]


_NW = 32
_PERW = _EP // _NW
_NJ = _PERW // 128
_NPS = _NP // 16


_CH = 64
_NC = _PERW // _CH


def _sc_gather_pair(d_width):
    mesh = plsc.VectorSubcoreMesh(core_axis_name="c", subcore_axis_name="s")

    @functools.partial(
        pl.kernel,
        mesh=mesh,
        out_type=[jax.ShapeDtypeStruct((_EP, d_width), _F32),
                  jax.ShapeDtypeStruct((_EP, d_width), _F32)],
        scratch_types=[
            pltpu.VMEM((_NC, _CH), jnp.int32),
            pltpu.VMEM((_NC, _CH), jnp.int32),
            pltpu.VMEM((_CH, d_width), _F32),
            pltpu.VMEM((_CH, d_width), _F32),
            pltpu.VMEM((_CH, d_width), _F32),
            pltpu.VMEM((_CH, d_width), _F32),
        ] + [pltpu.SemaphoreType.DMA] * 8,
    )
    def gather(tab_s, tab_d, idx_s, idx_d, out_s, out_d,
               iv_s, iv_d, bs0, bs1, bd0, bd1,
               gs0, gs1, gd0, gd1, ws0, ws1, wd0, wd1):
        wid = lax.axis_index("s") * 2 + lax.axis_index("c")
        base = wid * _PERW
        pltpu.sync_copy(idx_s.at[wid], iv_s)
        pltpu.sync_copy(idx_d.at[wid], iv_d)
        sb, db = (bs0, bs1), (bd0, bd1)
        gs, gd = (gs0, gs1), (gd0, gd1)
        ws, wd = (ws0, ws1), (wd0, wd1)
        pltpu.async_copy(tab_s.at[iv_s.at[0]], bs0, gs0)
        pltpu.async_copy(tab_d.at[iv_d.at[0]], bd0, gd0)

        def body(jj, carry):
            for t in range(2):
                j = jj * 2 + t
                u = 1 - t
                jn = j + 1

                @pl.when(jn < _NC)
                def _():
                    @pl.when(jn >= 2)
                    def _():
                        pltpu.make_async_copy(sb[u], out_s.at[pl.ds(0, _CH)], ws[u]).wait()
                        pltpu.make_async_copy(db[u], out_d.at[pl.ds(0, _CH)], wd[u]).wait()
                    pltpu.async_copy(tab_s.at[iv_s.at[jn]], sb[u], gs[u])
                    pltpu.async_copy(tab_d.at[iv_d.at[jn]], db[u], gd[u])

                off = pl.multiple_of(base + j * _CH, _CH)
                pltpu.make_async_copy(tab_s.at[iv_s.at[0]], sb[t], gs[t]).wait()
                pltpu.make_async_copy(tab_d.at[iv_d.at[0]], db[t], gd[t]).wait()
                pltpu.async_copy(sb[t], out_s.at[pl.ds(off, _CH)], ws[t])
                pltpu.async_copy(db[t], out_d.at[pl.ds(off, _CH)], wd[t])
            return carry

        lax.fori_loop(0, _NC // 2, body, 0)
        for t in range(2):
            pltpu.make_async_copy(sb[t], out_s.at[pl.ds(0, _CH)], ws[t]).wait()
            pltpu.make_async_copy(db[t], out_d.at[pl.ds(0, _CH)], wd[t]).wait()

    return gather


def _sc_scatter():
    mesh = plsc.VectorSubcoreMesh(core_axis_name="c", subcore_axis_name="s")

    @functools.partial(
        pl.kernel,
        mesh=mesh,
        out_type=jax.ShapeDtypeStruct((2 * _NP, 128), _F32),
        scratch_types=[
            pltpu.VMEM((64,), jnp.int32),
            pltpu.VMEM((64, 128), _F32),
            pltpu.VMEM_SHARED((_NP, 128), _F32),
        ],
    )
    def scat(r1_hbm, dsti_hbm, z128_hbm, aggh_out, ivc, rowbuf, acc):
        cid = lax.axis_index("c")
        sid = lax.axis_index("s")
        wid = sid * 2 + cid
        base = wid * _PERW
        pltpu.sync_copy(z128_hbm, rowbuf)
        for k in range(_NPS // 64):
            ns = pl.ds(sid * _NPS + k * 64, 64)
            pltpu.sync_copy(rowbuf, acc.at[ns])
        plsc.subcore_barrier()

        def body(j, carry):
            off = pl.multiple_of(base + j * 64, 64)
            pltpu.sync_copy(dsti_hbm.at[wid, j], ivc)
            pltpu.sync_copy(r1_hbm.at[pl.ds(off, 64)], rowbuf)
            pltpu.sync_copy(rowbuf, acc.at[ivc], add=True)
            return carry

        lax.fori_loop(0, 2 * _NJ, body, 0)
        plsc.subcore_barrier()
        for k in range(_NPS // 64):
            ns = pl.ds(sid * _NPS + k * 64, 64)
            oo = pl.ds(cid * _NP + sid * _NPS + k * 64, 64)
            pltpu.sync_copy(acc.at[ns], rowbuf)
            pltpu.sync_copy(rowbuf, aggh_out.at[oo])

    return scat


def _sc_degree():
    mesh = plsc.VectorSubcoreMesh(core_axis_name="c", subcore_axis_name="s")

    @functools.partial(
        pl.kernel,
        mesh=mesh,
        out_type=jax.ShapeDtypeStruct((2 * _NP, 128), _F32),
        scratch_types=[
            pltpu.VMEM((64,), jnp.int32),
            pltpu.VMEM((64, 128), _F32),
            pltpu.VMEM_SHARED((_NP, 128), _F32),
        ],
    )
    def degk(dsti_hbm, z128_hbm, ones_hbm, deg_out, ivc, buf, dacc):
        cid = lax.axis_index("c")
        sid = lax.axis_index("s")
        wid = sid * 2 + cid
        pltpu.sync_copy(z128_hbm, buf)
        for k in range(_NPS // 64):
            ns = pl.ds(sid * _NPS + k * 64, 64)
            pltpu.sync_copy(buf, dacc.at[ns])
        pltpu.sync_copy(ones_hbm, buf)
        plsc.subcore_barrier()

        def body(j, carry):
            pltpu.sync_copy(dsti_hbm.at[wid, j], ivc)
            pltpu.sync_copy(buf, dacc.at[ivc], add=True)
            return carry

        lax.fori_loop(0, 2 * _NJ, body, 0)
        plsc.subcore_barrier()
        for k in range(_NPS // 64):
            ns = pl.ds(sid * _NPS + k * 64, 64)
            oo = pl.ds(cid * _NP + sid * _NPS + k * 64, 64)
            pltpu.sync_copy(dacc.at[ns], buf)
            pltpu.sync_copy(buf, deg_out.at[oo])

    return degk


def _full(r, c):
    return pl.BlockSpec((r, c), lambda i: (0, 0))


def _rows(b, c):
    return pl.BlockSpec((b, c), lambda i: (i, 0))


_SEL = np.zeros((32, 8), dtype=np.float32)
for _p in range(8):
    for _j in range(3):
        _SEL[3 * _p + _j, _p] = 1.0


def kernel(node_features, rigid_rots, rigid_trans, edge_features, res_mask,
           noising_mask, W_pts, b_pts, W_m1, b_m1, W_m2, b_m2, W_n1, b_n1,
           W_n2, b_n2, g_s, b_s, W_e1, b_e1, W_e2, b_e2, g_z, b_z, W_bb,
           b_bb, edge_index):
    pad_idx = _N + (jnp.arange(_EP - _E, dtype=jnp.int32) % (_NP - _N))
    src_p = jnp.concatenate([edge_index[0], pad_idx])
    dst_p = jnp.concatenate([edge_index[1], pad_idx])
    nf = jnp.pad(node_features, ((0, _NP - _N), (0, 0)))
    ef = jnp.pad(edge_features, ((0, _EP - _E), (0, 0)))
    rotsT = jnp.pad(rigid_rots.reshape(_N, 9), ((0, _NP - _N), (0, 7))).T
    trT = jnp.pad(rigid_trans, ((0, _NP - _N), (0, 5))).T
    rm = jnp.pad(res_mask, (0, _NP - _N)).reshape(_NP, 1)
    nm = jnp.pad(noising_mask, (0, _NP - _N)).reshape(_NP, 1)
    wptsP = jnp.pad(W_pts, ((0, 0), (0, 8)))
    bptsP = jnp.pad(b_pts, (0, 8)).reshape(1, 32)
    wm1a, wm1b, wef1, wd1 = W_m1[:256], W_m1[256:512], W_m1[512:640], W_m1[640:648]
    we1e, we1s, we1d = W_e1[:128], W_e1[128:384], W_e1[384:640]
    wn1a, wn1b = W_n1[:256], W_n1[256:512]
    wbbP = jnp.pad(W_bb, ((0, 0), (0, 2)))
    bbbP = jnp.pad(b_bb, (0, 2)).reshape(1, 8)
    sel = jnp.asarray(_SEL)
    r2 = lambda v: v.reshape(1, -1)

    a_tab, b_tab, pts = pl.pallas_call(
        _nodeprep_body,
        grid=(_NP // 1024,),
        in_specs=[_rows(1024, 256), _full(256, 32), _full(1, 32),
                  _full(256, 128), _full(256, 128)],
        out_specs=[_rows(1024, 128), _rows(1024, 128), _rows(1024, 32)],
        out_shape=[jax.ShapeDtypeStruct((_NP, 128), _F32),
                   jax.ShapeDtypeStruct((_NP, 128), _F32),
                   jax.ShapeDtypeStruct((_NP, 32), _F32)],
    )(nf, wptsP, bptsP, wm1a, wm1b)

    gpT = pl.pallas_call(
        _gpt_body,
        grid=(_NP // 2048,),
        in_specs=[_rows_t(32, 2048), _rows_t(16, 2048), _rows_t(8, 2048)],
        out_specs=_rows_t(32, 2048),
        out_shape=jax.ShapeDtypeStruct((32, _NP), _F32),
    )(pts.T, rotsT, trT)
    gp_nodes = gpT.T
    padc = jnp.zeros((_NP, 96), _F32)
    t_src = jnp.concatenate([a_tab, gp_nodes, padc], axis=1)
    t_dst = jnp.concatenate([b_tab, gp_nodes, padc], axis=1)

    src3 = src_p.reshape(_NW, _NC, _CH)
    dst3 = dst_p.reshape(_NW, _NC, _CH)
    gs1, gd1 = _sc_gather_pair(256)(t_src, t_dst, src3, dst3)

    r1 = pl.pallas_call(
        _msg_body,
        grid=(_EP // 2048,),
        in_specs=[_rows(2048, 256), _rows(2048, 256), _rows(2048, 128),
                  _full(32, 8), _full(8, 128), _full(128, 128), _full(1, 128)],
        out_specs=_rows(2048, 128),
        out_shape=jax.ShapeDtypeStruct((_EP, 128), _F32),
    )(gs1, gd1, ef, sel, wd1, wef1, r2(b_m1))

    z64 = jnp.zeros((64, 128), _F32)
    agg2 = _sc_scatter()(r1, dst3, z64)
    degf = _sc_degree()(dst3, z64, jnp.ones((64, 128), _F32))
    aggh = agg2[:_NP] + agg2[_NP:]
    deg16 = (degf[:_NP] + degf[_NP:])[:, :16]

    s, as_tab, bs_tab, upd = pl.pallas_call(
        _node_body,
        grid=(_NP // 1024,),
        in_specs=[_rows(1024, 256), _rows(1024, 128), _rows(1024, 16),
                  _full(128, 256), _full(1, 256), _full(256, 128), _full(256, 128),
                  _full(1, 128), _full(128, 256), _full(1, 256), _full(1, 256),
                  _full(1, 256), _rows(1024, 1), _rows(1024, 1),
                  _full(256, 128), _full(256, 128), _full(256, 8), _full(1, 8)],
        out_specs=[_rows(1024, 256), _rows(1024, 128), _rows(1024, 128),
                   _rows(1024, 8)],
        out_shape=[jax.ShapeDtypeStruct((_NP, 256), _F32),
                   jax.ShapeDtypeStruct((_NP, 128), _F32),
                   jax.ShapeDtypeStruct((_NP, 128), _F32),
                   jax.ShapeDtypeStruct((_NP, 8), _F32)],
    )(nf, aggh, deg16, W_m2, r2(b_m2), wn1a, wn1b, r2(b_n1), W_n2, r2(b_n2),
      r2(g_s), r2(b_s), rm, nm, we1s, we1d, wbbP, bbbP)

    nrotT, ntrT = pl.pallas_call(
        _bb_body,
        grid=(_NP // 2048,),
        in_specs=[_rows_t(8, 2048), _rows_t(16, 2048), _rows_t(8, 2048)],
        out_specs=[_rows_t(16, 2048), _rows_t(8, 2048)],
        out_shape=[jax.ShapeDtypeStruct((16, _NP), _F32),
                   jax.ShapeDtypeStruct((8, _NP), _F32)],
    )(upd.T, rotsT, trT)

    gs2, gd2 = _sc_gather_pair(128)(as_tab, bs_tab, src3, dst3)

    z = pl.pallas_call(
        _edge_body,
        grid=(_EP // 2048,),
        in_specs=[_rows(2048, 128), _rows(2048, 128), _rows(2048, 128),
                  _full(128, 128), _full(1, 128), _full(128, 128), _full(1, 128),
                  _full(1, 128), _full(1, 128)],
        out_specs=_rows(2048, 128),
        out_shape=jax.ShapeDtypeStruct((_EP, 128), _F32),
    )(ef, gs2, gd2, we1e, r2(b_e1), W_e2, r2(b_e2), r2(g_z), r2(b_z))

    s_out = s[:_N]
    new_rots = nrotT.T[:_N, :9].reshape(_N, 3, 3)
    new_trans = ntrT.T[:_N, :3]
    return (s_out, new_rots, new_trans, z[:_E])


def _rows_t(r, b):
    return pl.BlockSpec((r, b), lambda i: (0, i))


# speedup vs baseline: 4.9663x; 1.0881x over previous
import functools

import jax
import jax.numpy as jnp
import numpy as np
from jax import lax
from jax.experimental import pallas as pl
from jax.experimental.pallas import tpu as pltpu
from jax.experimental.pallas import tpu_sc as plsc

_N = 10000
_E = 160000
_NP = 10240
_EP = 163840

_F32 = jnp.float32


def _nodeprep_body(nf_ref, wpts_ref, bpts_ref, wa_ref, wb_ref,
                   a_ref, b_ref, pts_ref):
    nf = nf_ref[...]
    a_ref[...] = jnp.dot(nf, wa_ref[...], preferred_element_type=_F32)
    b_ref[...] = jnp.dot(nf, wb_ref[...], preferred_element_type=_F32)
    pts_ref[...] = jnp.dot(nf, wpts_ref[...], preferred_element_type=_F32) + bpts_ref[...]


def _gpt_body(pts_ref, rot_ref, tr_ref, out_ref):
    pts = pts_ref[...]
    rot = rot_ref[...]
    tr = tr_ref[...]
    rows = []
    for p in range(8):
        for i in range(3):
            acc = tr[i:i + 1, :]
            for j in range(3):
                acc = acc + rot[3 * i + j:3 * i + j + 1, :] * pts[3 * p + j:3 * p + j + 1, :]
            rows.append(acc)
    zero = jnp.zeros_like(rows[0])
    out_ref[...] = jnp.concatenate(rows + [zero] * 8, axis=0)


def _msg_body(gs_ref, gd_ref, ef_ref, sel_ref, wd_ref, wef_ref, bm1_ref, r1_ref):
    gs = gs_ref[...]
    gd = gd_ref[...]
    dg = gs[:, 128:160] - gd[:, 128:160]
    d2 = jnp.dot(dg * dg, sel_ref[...], preferred_element_type=_F32)
    d = jnp.sqrt(d2 + 1e-8)
    h = gs[:, :128] + gd[:, :128]
    h = h + jnp.dot(ef_ref[...], wef_ref[...], preferred_element_type=_F32)
    h = h + jnp.dot(d, wd_ref[...], preferred_element_type=_F32)
    r1_ref[...] = jnp.maximum(h + bm1_ref[...], 0.0)


def _node_body(nf_ref, aggh_ref, deg_ref, wm2_ref, bm2_ref, wn1a_ref, wn1b_ref,
               bn1_ref, wn2_ref, bn2_ref, gsn_ref, bsn_ref, rm_ref, nm_ref,
               we1s_ref, we1d_ref, wbb_ref, bbb_ref,
               s_ref, as_ref, bsd_ref, upd_ref):
    nf = nf_ref[...]
    deg = deg_ref[:, 0:1]
    agg = jnp.dot(aggh_ref[...], wm2_ref[...], preferred_element_type=_F32)
    agg = agg / jnp.maximum(deg, 1.0) + bm2_ref[...] * jnp.minimum(deg, 1.0)
    h = jnp.dot(nf, wn1a_ref[...], preferred_element_type=_F32)
    h = h + jnp.dot(agg, wn1b_ref[...], preferred_element_type=_F32)
    h = jnp.maximum(h + bn1_ref[...], 0.0)
    su = nf + jnp.dot(h, wn2_ref[...], preferred_element_type=_F32) + bn2_ref[...]
    mu = jnp.mean(su, axis=1, keepdims=True)
    var = jnp.mean((su - mu) * (su - mu), axis=1, keepdims=True)
    s = (su - mu) / jnp.sqrt(var + 1e-5) * gsn_ref[...] + bsn_ref[...]
    s = s * rm_ref[...]
    s_ref[...] = s
    as_ref[...] = jnp.dot(s, we1s_ref[...], preferred_element_type=_F32)
    bsd_ref[...] = jnp.dot(s, we1d_ref[...], preferred_element_type=_F32)
    sm = s * nm_ref[...]
    upd_ref[...] = jnp.dot(sm, wbb_ref[...], preferred_element_type=_F32) * nm_ref[...]


def _bb_body(upd_ref, rot_ref, tr_ref, nrot_ref, ntr_ref):
    u = upd_ref[...]
    b = u[0:1]; c = u[1:2]; d = u[2:3]
    inv = lax.rsqrt(1.0 + b * b + c * c + d * d)
    w = inv; x = b * inv; y = c * inv; z = d * inv
    ru = [1 - 2 * (y * y + z * z), 2 * (x * y - w * z), 2 * (x * z + w * y),
          2 * (x * y + w * z), 1 - 2 * (x * x + z * z), 2 * (y * z - w * x),
          2 * (x * z - w * y), 2 * (y * z + w * x), 1 - 2 * (x * x + y * y)]
    rot = rot_ref[...]
    out_rows = []
    for i in range(3):
        for k in range(3):
            acc = rot[3 * i + 0:3 * i + 1] * ru[0 * 3 + k]
            acc = acc + rot[3 * i + 1:3 * i + 2] * ru[1 * 3 + k]
            acc = acc + rot[3 * i + 2:3 * i + 3] * ru[2 * 3 + k]
            out_rows.append(acc)
    zero = jnp.zeros_like(out_rows[0])
    nrot_ref[...] = jnp.concatenate(out_rows + [zero] * 7, axis=0)
    tr = tr_ref[...]
    trows = []
    for i in range(3):
        acc = tr[i:i + 1]
        for j in range(3):
            acc = acc + rot[3 * i + j:3 * i + j + 1] * u[3 + j:4 + j]
        trows.append(acc)
    ntr_ref[...] = jnp.concatenate(trows + [zero] * 5, axis=0)


def _edge_body(ef_ref, gs_ref, gd_ref, we1e_ref, be1_ref, we2_ref, be2_ref,
               gz_ref, bz_ref, z_ref):
    ef = ef_ref[...]
    h = jnp.dot(ef, we1e_ref[...], preferred_element_type=_F32)
    h = jnp.maximum(h + gs_ref[...] + gd_ref[...] + be1_ref[...], 0.0)
    zz = ef + jnp.dot(h, we2_ref[...], preferred_element_type=_F32) + be2_ref[...]
    mu = jnp.mean(zz, axis=1, keepdims=True)
    var = jnp.mean((zz - mu) * (zz - mu), axis=1, keepdims=True)
    z_ref[...] = (zz - mu) / jnp.sqrt(var + 1e-5) * gz_ref[...] + bz_ref[...]


_NW = 32
_PERW = _EP // _NW
_NJ = _PERW // 128
_NPS = _NP // 16


_CH = 64
_NC = _PERW // _CH


def _sc_gather_pair(d_width):
    mesh = plsc.VectorSubcoreMesh(core_axis_name="c", subcore_axis_name="s")

    @functools.partial(
        pl.kernel,
        mesh=mesh,
        out_type=[jax.ShapeDtypeStruct((_EP, d_width), _F32),
                  jax.ShapeDtypeStruct((_EP, d_width), _F32)],
        scratch_types=[
            pltpu.VMEM((_NC, _CH), jnp.int32),
            pltpu.VMEM((_NC, _CH), jnp.int32),
            pltpu.VMEM((_CH, d_width), _F32),
            pltpu.VMEM((_CH, d_width), _F32),
            pltpu.VMEM((_CH, d_width), _F32),
            pltpu.VMEM((_CH, d_width), _F32),
        ] + [pltpu.SemaphoreType.DMA] * 8,
    )
    def gather(tab_s, tab_d, idx_s, idx_d, out_s, out_d,
               iv_s, iv_d, bs0, bs1, bd0, bd1,
               gs0, gs1, gd0, gd1, ws0, ws1, wd0, wd1):
        wid = lax.axis_index("s") * 2 + lax.axis_index("c")
        base = wid * _PERW
        pltpu.sync_copy(idx_s.at[wid], iv_s)
        pltpu.sync_copy(idx_d.at[wid], iv_d)
        sb, db = (bs0, bs1), (bd0, bd1)
        gs, gd = (gs0, gs1), (gd0, gd1)
        ws, wd = (ws0, ws1), (wd0, wd1)
        pltpu.async_copy(tab_s.at[iv_s.at[0]], bs0, gs0)
        pltpu.async_copy(tab_d.at[iv_d.at[0]], bd0, gd0)

        def body(jj, carry):
            for t in range(2):
                j = jj * 2 + t
                u = 1 - t
                jn = j + 1

                @pl.when(jn < _NC)
                def _():
                    @pl.when(jn >= 2)
                    def _():
                        pltpu.make_async_copy(sb[u], out_s.at[pl.ds(0, _CH)], ws[u]).wait()
                        pltpu.make_async_copy(db[u], out_d.at[pl.ds(0, _CH)], wd[u]).wait()
                    pltpu.async_copy(tab_s.at[iv_s.at[jn]], sb[u], gs[u])
                    pltpu.async_copy(tab_d.at[iv_d.at[jn]], db[u], gd[u])

                off = pl.multiple_of(base + j * _CH, _CH)
                pltpu.make_async_copy(tab_s.at[iv_s.at[0]], sb[t], gs[t]).wait()
                pltpu.make_async_copy(tab_d.at[iv_d.at[0]], db[t], gd[t]).wait()
                pltpu.async_copy(sb[t], out_s.at[pl.ds(off, _CH)], ws[t])
                pltpu.async_copy(db[t], out_d.at[pl.ds(off, _CH)], wd[t])
            return carry

        lax.fori_loop(0, _NC // 2, body, 0)
        for t in range(2):
            pltpu.make_async_copy(sb[t], out_s.at[pl.ds(0, _CH)], ws[t]).wait()
            pltpu.make_async_copy(db[t], out_d.at[pl.ds(0, _CH)], wd[t]).wait()

    return gather


def _sc_scatter():
    mesh = plsc.VectorSubcoreMesh(core_axis_name="c", subcore_axis_name="s")

    @functools.partial(
        pl.kernel,
        mesh=mesh,
        out_type=jax.ShapeDtypeStruct((2 * _NP, 128), _F32),
        scratch_types=[
            pltpu.VMEM((_CH,), jnp.int32),
            pltpu.VMEM((_CH,), jnp.int32),
            pltpu.VMEM((_CH, 128), _F32),
            pltpu.VMEM((_CH, 128), _F32),
            pltpu.VMEM_SHARED((_NP, 128), _F32),
        ] + [pltpu.SemaphoreType.DMA] * 6,
    )
    def scat(r1_hbm, dsti_hbm, z128_hbm, aggh_out,
             iv0, iv1, rb0, rb1, acc, li0, li1, lr0, lr1, a0, a1):
        cid = lax.axis_index("c")
        sid = lax.axis_index("s")
        wid = sid * 2 + cid
        base = wid * _PERW
        pltpu.sync_copy(z128_hbm, rb0)
        for k in range(_NPS // 64):
            ns = pl.ds(sid * _NPS + k * 64, 64)
            pltpu.sync_copy(rb0, acc.at[ns])
        plsc.subcore_barrier()
        iv, rb = (iv0, iv1), (rb0, rb1)
        li, lr, ad = (li0, li1), (lr0, lr1), (a0, a1)
        pltpu.async_copy(dsti_hbm.at[wid, 0], iv0, li0)
        pltpu.async_copy(r1_hbm.at[pl.ds(base, _CH)], rb0, lr0)

        def body(jj, carry):
            for t in range(2):
                j = jj * 2 + t
                u = 1 - t
                jn = j + 1

                @pl.when(jn < _NC)
                def _():
                    @pl.when(jn >= 2)
                    def _():
                        pltpu.make_async_copy(rb[u], acc.at[iv[u]], ad[u]).wait()
                    offn = pl.multiple_of(base + jn * _CH, _CH)
                    pltpu.async_copy(dsti_hbm.at[wid, jn], iv[u], li[u])
                    pltpu.async_copy(r1_hbm.at[pl.ds(offn, _CH)], rb[u], lr[u])

                pltpu.make_async_copy(dsti_hbm.at[wid, 0], iv[t], li[t]).wait()
                pltpu.make_async_copy(r1_hbm.at[pl.ds(base, _CH)], rb[t], lr[t]).wait()
                pltpu.async_copy(rb[t], acc.at[iv[t]], ad[t], add=True)
            return carry

        lax.fori_loop(0, _NC // 2, body, 0)
        for t in range(2):
            pltpu.make_async_copy(rb[t], acc.at[iv[t]], ad[t]).wait()
        plsc.subcore_barrier()
        for k in range(_NPS // 64):
            ns = pl.ds(sid * _NPS + k * 64, 64)
            oo = pl.ds(cid * _NP + sid * _NPS + k * 64, 64)
            pltpu.sync_copy(acc.at[ns], rb0)
            pltpu.sync_copy(rb0, aggh_out.at[oo])

    return scat


def _sc_degree():
    mesh = plsc.VectorSubcoreMesh(core_axis_name="c", subcore_axis_name="s")

    @functools.partial(
        pl.kernel,
        mesh=mesh,
        out_type=jax.ShapeDtypeStruct((2 * _NP, 128), _F32),
        scratch_types=[
            pltpu.VMEM((_CH,), jnp.int32),
            pltpu.VMEM((_CH,), jnp.int32),
            pltpu.VMEM((_CH, 128), _F32),
            pltpu.VMEM_SHARED((_NP, 128), _F32),
        ] + [pltpu.SemaphoreType.DMA] * 4,
    )
    def degk(dsti_hbm, z128_hbm, ones_hbm, deg_out,
             iv0, iv1, buf, dacc, li0, li1, a0, a1):
        cid = lax.axis_index("c")
        sid = lax.axis_index("s")
        wid = sid * 2 + cid
        pltpu.sync_copy(z128_hbm, buf)
        for k in range(_NPS // 64):
            ns = pl.ds(sid * _NPS + k * 64, 64)
            pltpu.sync_copy(buf, dacc.at[ns])
        pltpu.sync_copy(ones_hbm, buf)
        plsc.subcore_barrier()
        iv, li, ad = (iv0, iv1), (li0, li1), (a0, a1)
        pltpu.async_copy(dsti_hbm.at[wid, 0], iv0, li0)

        def body(jj, carry):
            for t in range(2):
                j = jj * 2 + t
                u = 1 - t
                jn = j + 1

                @pl.when(jn < _NC)
                def _():
                    @pl.when(jn >= 2)
                    def _():
                        pltpu.make_async_copy(buf, dacc.at[iv[u]], ad[u]).wait()
                    pltpu.async_copy(dsti_hbm.at[wid, jn], iv[u], li[u])

                pltpu.make_async_copy(dsti_hbm.at[wid, 0], iv[t], li[t]).wait()
                pltpu.async_copy(buf, dacc.at[iv[t]], ad[t], add=True)
            return carry

        lax.fori_loop(0, _NC // 2, body, 0)
        for t in range(2):
            pltpu.make_async_copy(buf, dacc.at[iv[t]], ad[t]).wait()
        plsc.subcore_barrier()
        for k in range(_NPS // 64):
            ns = pl.ds(sid * _NPS + k * 64, 64)
            oo = pl.ds(cid * _NP + sid * _NPS + k * 64, 64)
            pltpu.sync_copy(dacc.at[ns], buf)
            pltpu.sync_copy(buf, deg_out.at[oo])

    return degk


def _full(r, c):
    return pl.BlockSpec((r, c), lambda i: (0, 0))


def _rows(b, c):
    return pl.BlockSpec((b, c), lambda i: (i, 0))


_SEL = np.zeros((32, 8), dtype=np.float32)
for _p in range(8):
    for _j in range(3):
        _SEL[3 * _p + _j, _p] = 1.0


def kernel(node_features, rigid_rots, rigid_trans, edge_features, res_mask,
           noising_mask, W_pts, b_pts, W_m1, b_m1, W_m2, b_m2, W_n1, b_n1,
           W_n2, b_n2, g_s, b_s, W_e1, b_e1, W_e2, b_e2, g_z, b_z, W_bb,
           b_bb, edge_index):
    pad_idx = _N + (jnp.arange(_EP - _E, dtype=jnp.int32) % (_NP - _N))
    src_p = jnp.concatenate([edge_index[0], pad_idx])
    dst_p = jnp.concatenate([edge_index[1], pad_idx])
    nf = jnp.pad(node_features, ((0, _NP - _N), (0, 0)))
    ef = jnp.pad(edge_features, ((0, _EP - _E), (0, 0)))
    rotsT = jnp.pad(rigid_rots.reshape(_N, 9), ((0, _NP - _N), (0, 7))).T
    trT = jnp.pad(rigid_trans, ((0, _NP - _N), (0, 5))).T
    rm = jnp.pad(res_mask, (0, _NP - _N)).reshape(_NP, 1)
    nm = jnp.pad(noising_mask, (0, _NP - _N)).reshape(_NP, 1)
    wptsP = jnp.pad(W_pts, ((0, 0), (0, 8)))
    bptsP = jnp.pad(b_pts, (0, 8)).reshape(1, 32)
    wm1a, wm1b, wef1, wd1 = W_m1[:256], W_m1[256:512], W_m1[512:640], W_m1[640:648]
    we1e, we1s, we1d = W_e1[:128], W_e1[128:384], W_e1[384:640]
    wn1a, wn1b = W_n1[:256], W_n1[256:512]
    wbbP = jnp.pad(W_bb, ((0, 0), (0, 2)))
    bbbP = jnp.pad(b_bb, (0, 2)).reshape(1, 8)
    sel = jnp.asarray(_SEL)
    r2 = lambda v: v.reshape(1, -1)

    a_tab, b_tab, pts = pl.pallas_call(
        _nodeprep_body,
        grid=(_NP // 1024,),
        in_specs=[_rows(1024, 256), _full(256, 32), _full(1, 32),
                  _full(256, 128), _full(256, 128)],
        out_specs=[_rows(1024, 128), _rows(1024, 128), _rows(1024, 32)],
        out_shape=[jax.ShapeDtypeStruct((_NP, 128), _F32),
                   jax.ShapeDtypeStruct((_NP, 128), _F32),
                   jax.ShapeDtypeStruct((_NP, 32), _F32)],
    )(nf, wptsP, bptsP, wm1a, wm1b)

    gpT = pl.pallas_call(
        _gpt_body,
        grid=(_NP // 2048,),
        in_specs=[_rows_t(32, 2048), _rows_t(16, 2048), _rows_t(8, 2048)],
        out_specs=_rows_t(32, 2048),
        out_shape=jax.ShapeDtypeStruct((32, _NP), _F32),
    )(pts.T, rotsT, trT)
    gp_nodes = gpT.T
    padc = jnp.zeros((_NP, 96), _F32)
    t_src = jnp.concatenate([a_tab, gp_nodes, padc], axis=1)
    t_dst = jnp.concatenate([b_tab, gp_nodes, padc], axis=1)

    src3 = src_p.reshape(_NW, _NC, _CH)
    dst3 = dst_p.reshape(_NW, _NC, _CH)
    gs1, gd1 = _sc_gather_pair(256)(t_src, t_dst, src3, dst3)

    r1 = pl.pallas_call(
        _msg_body,
        grid=(_EP // 2048,),
        in_specs=[_rows(2048, 256), _rows(2048, 256), _rows(2048, 128),
                  _full(32, 8), _full(8, 128), _full(128, 128), _full(1, 128)],
        out_specs=_rows(2048, 128),
        out_shape=jax.ShapeDtypeStruct((_EP, 128), _F32),
    )(gs1, gd1, ef, sel, wd1, wef1, r2(b_m1))

    z64 = jnp.zeros((64, 128), _F32)
    agg2 = _sc_scatter()(r1, dst3, z64)
    degf = _sc_degree()(dst3, z64, jnp.ones((64, 128), _F32))
    aggh = agg2[:_NP] + agg2[_NP:]
    deg16 = (degf[:_NP] + degf[_NP:])[:, :16]

    s, as_tab, bs_tab, upd = pl.pallas_call(
        _node_body,
        grid=(_NP // 1024,),
        in_specs=[_rows(1024, 256), _rows(1024, 128), _rows(1024, 16),
                  _full(128, 256), _full(1, 256), _full(256, 128), _full(256, 128),
                  _full(1, 128), _full(128, 256), _full(1, 256), _full(1, 256),
                  _full(1, 256), _rows(1024, 1), _rows(1024, 1),
                  _full(256, 128), _full(256, 128), _full(256, 8), _full(1, 8)],
        out_specs=[_rows(1024, 256), _rows(1024, 128), _rows(1024, 128),
                   _rows(1024, 8)],
        out_shape=[jax.ShapeDtypeStruct((_NP, 256), _F32),
                   jax.ShapeDtypeStruct((_NP, 128), _F32),
                   jax.ShapeDtypeStruct((_NP, 128), _F32),
                   jax.ShapeDtypeStruct((_NP, 8), _F32)],
    )(nf, aggh, deg16, W_m2, r2(b_m2), wn1a, wn1b, r2(b_n1), W_n2, r2(b_n2),
      r2(g_s), r2(b_s), rm, nm, we1s, we1d, wbbP, bbbP)

    nrotT, ntrT = pl.pallas_call(
        _bb_body,
        grid=(_NP // 2048,),
        in_specs=[_rows_t(8, 2048), _rows_t(16, 2048), _rows_t(8, 2048)],
        out_specs=[_rows_t(16, 2048), _rows_t(8, 2048)],
        out_shape=[jax.ShapeDtypeStruct((16, _NP), _F32),
                   jax.ShapeDtypeStruct((8, _NP), _F32)],
    )(upd.T, rotsT, trT)

    gs2, gd2 = _sc_gather_pair(128)(as_tab, bs_tab, src3, dst3)

    z = pl.pallas_call(
        _edge_body,
        grid=(_EP // 2048,),
        in_specs=[_rows(2048, 128), _rows(2048, 128), _rows(2048, 128),
                  _full(128, 128), _full(1, 128), _full(128, 128), _full(1, 128),
                  _full(1, 128), _full(1, 128)],
        out_specs=_rows(2048, 128),
        out_shape=jax.ShapeDtypeStruct((_EP, 128), _F32),
    )(ef, gs2, gd2, we1e, r2(b_e1), W_e2, r2(b_e2), r2(g_z), r2(b_z))

    s_out = s[:_N]
    new_rots = nrotT.T[:_N, :9].reshape(_N, 3, 3)
    new_trans = ntrT.T[:_N, :3]
    return (s_out, new_rots, new_trans, z[:_E])


def _rows_t(r, b):
    return pl.BlockSpec((r, b), lambda i: (0, i))


# speedup vs baseline: 4.9708x; 1.0009x over previous
import functools

import jax
import jax.numpy as jnp
import numpy as np
from jax import lax
from jax.experimental import pallas as pl
from jax.experimental.pallas import tpu as pltpu
from jax.experimental.pallas import tpu_sc as plsc

_N = 10000
_E = 160000
_NP = 10240
_EP = 163840

_F32 = jnp.float32


def _nodeprep_body(nf_ref, wpts_ref, bpts_ref, wa_ref, wb_ref,
                   a_ref, b_ref, pts_ref):
    nf = nf_ref[...]
    a_ref[...] = jnp.dot(nf, wa_ref[...], preferred_element_type=_F32)
    b_ref[...] = jnp.dot(nf, wb_ref[...], preferred_element_type=_F32)
    pts_ref[...] = jnp.dot(nf, wpts_ref[...], preferred_element_type=_F32) + bpts_ref[...]


def _gpt_body(pts_ref, rot_ref, tr_ref, out_ref):
    pts = pts_ref[...]
    rot = rot_ref[...]
    tr = tr_ref[...]
    rows = []
    for p in range(8):
        for i in range(3):
            acc = tr[i:i + 1, :]
            for j in range(3):
                acc = acc + rot[3 * i + j:3 * i + j + 1, :] * pts[3 * p + j:3 * p + j + 1, :]
            rows.append(acc)
    zero = jnp.zeros_like(rows[0])
    out_ref[...] = jnp.concatenate(rows + [zero] * 8, axis=0)


def _msg_body(gs_ref, gd_ref, ef_ref, sel_ref, wd_ref, wef_ref, bm1_ref, r1_ref):
    gs = gs_ref[...]
    gd = gd_ref[...]
    dg = gs[:, 128:160] - gd[:, 128:160]
    d2 = jnp.dot(dg * dg, sel_ref[...], preferred_element_type=_F32)
    d = jnp.sqrt(d2 + 1e-8)
    h = gs[:, :128] + gd[:, :128]
    h = h + jnp.dot(ef_ref[...], wef_ref[...], preferred_element_type=_F32)
    h = h + jnp.dot(d, wd_ref[...], preferred_element_type=_F32)
    r1_ref[...] = jnp.maximum(h + bm1_ref[...], 0.0)


def _node_body(nf_ref, aggh_ref, deg_ref, wm2_ref, bm2_ref, wn1a_ref, wn1b_ref,
               bn1_ref, wn2_ref, bn2_ref, gsn_ref, bsn_ref, rm_ref, nm_ref,
               we1s_ref, we1d_ref, wbb_ref, bbb_ref,
               s_ref, as_ref, bsd_ref, upd_ref):
    nf = nf_ref[...]
    deg = deg_ref[:, 0:1]
    agg = jnp.dot(aggh_ref[...], wm2_ref[...], preferred_element_type=_F32)
    agg = agg / jnp.maximum(deg, 1.0) + bm2_ref[...] * jnp.minimum(deg, 1.0)
    h = jnp.dot(nf, wn1a_ref[...], preferred_element_type=_F32)
    h = h + jnp.dot(agg, wn1b_ref[...], preferred_element_type=_F32)
    h = jnp.maximum(h + bn1_ref[...], 0.0)
    su = nf + jnp.dot(h, wn2_ref[...], preferred_element_type=_F32) + bn2_ref[...]
    mu = jnp.mean(su, axis=1, keepdims=True)
    var = jnp.mean((su - mu) * (su - mu), axis=1, keepdims=True)
    s = (su - mu) / jnp.sqrt(var + 1e-5) * gsn_ref[...] + bsn_ref[...]
    s = s * rm_ref[...]
    s_ref[...] = s
    as_ref[...] = jnp.dot(s, we1s_ref[...], preferred_element_type=_F32)
    bsd_ref[...] = jnp.dot(s, we1d_ref[...], preferred_element_type=_F32)
    sm = s * nm_ref[...]
    upd_ref[...] = jnp.dot(sm, wbb_ref[...], preferred_element_type=_F32) * nm_ref[...]


def _bb_body(upd_ref, rot_ref, tr_ref, nrot_ref, ntr_ref):
    u = upd_ref[...]
    b = u[0:1]; c = u[1:2]; d = u[2:3]
    inv = lax.rsqrt(1.0 + b * b + c * c + d * d)
    w = inv; x = b * inv; y = c * inv; z = d * inv
    ru = [1 - 2 * (y * y + z * z), 2 * (x * y - w * z), 2 * (x * z + w * y),
          2 * (x * y + w * z), 1 - 2 * (x * x + z * z), 2 * (y * z - w * x),
          2 * (x * z - w * y), 2 * (y * z + w * x), 1 - 2 * (x * x + y * y)]
    rot = rot_ref[...]
    out_rows = []
    for i in range(3):
        for k in range(3):
            acc = rot[3 * i + 0:3 * i + 1] * ru[0 * 3 + k]
            acc = acc + rot[3 * i + 1:3 * i + 2] * ru[1 * 3 + k]
            acc = acc + rot[3 * i + 2:3 * i + 3] * ru[2 * 3 + k]
            out_rows.append(acc)
    zero = jnp.zeros_like(out_rows[0])
    nrot_ref[...] = jnp.concatenate(out_rows + [zero] * 7, axis=0)
    tr = tr_ref[...]
    trows = []
    for i in range(3):
        acc = tr[i:i + 1]
        for j in range(3):
            acc = acc + rot[3 * i + j:3 * i + j + 1] * u[3 + j:4 + j]
        trows.append(acc)
    ntr_ref[...] = jnp.concatenate(trows + [zero] * 5, axis=0)


def _edge_body(ef_ref, gs_ref, gd_ref, we1e_ref, be1_ref, we2_ref, be2_ref,
               gz_ref, bz_ref, z_ref):
    ef = ef_ref[...]
    h = jnp.dot(ef, we1e_ref[...], preferred_element_type=_F32)
    h = jnp.maximum(h + gs_ref[...] + gd_ref[...] + be1_ref[...], 0.0)
    zz = ef + jnp.dot(h, we2_ref[...], preferred_element_type=_F32) + be2_ref[...]
    mu = jnp.mean(zz, axis=1, keepdims=True)
    var = jnp.mean((zz - mu) * (zz - mu), axis=1, keepdims=True)
    z_ref[...] = (zz - mu) / jnp.sqrt(var + 1e-5) * gz_ref[...] + bz_ref[...]


_NW = 32
_PERW = _EP // _NW
_NPS = _NP // 16


_CH = 64
_NC = _PERW // _CH


def _sc_gather_pair(d_width):
    mesh = plsc.VectorSubcoreMesh(core_axis_name="c", subcore_axis_name="s")

    @functools.partial(
        pl.kernel,
        mesh=mesh,
        out_type=[jax.ShapeDtypeStruct((_EP, d_width), _F32),
                  jax.ShapeDtypeStruct((_EP, d_width), _F32)],
        scratch_types=[
            pltpu.VMEM((_NC, _CH), jnp.int32),
            pltpu.VMEM((_NC, _CH), jnp.int32),
            pltpu.VMEM((_CH, d_width), _F32),
            pltpu.VMEM((_CH, d_width), _F32),
            pltpu.VMEM((_CH, d_width), _F32),
            pltpu.VMEM((_CH, d_width), _F32),
        ] + [pltpu.SemaphoreType.DMA] * 8,
    )
    def gather(tab_s, tab_d, idx_s, idx_d, out_s, out_d,
               iv_s, iv_d, bs0, bs1, bd0, bd1,
               gs0, gs1, gd0, gd1, ws0, ws1, wd0, wd1):
        wid = lax.axis_index("s") * 2 + lax.axis_index("c")
        base = wid * _PERW
        pltpu.sync_copy(idx_s.at[wid], iv_s)
        pltpu.sync_copy(idx_d.at[wid], iv_d)
        sb, db = (bs0, bs1), (bd0, bd1)
        gs, gd = (gs0, gs1), (gd0, gd1)
        ws, wd = (ws0, ws1), (wd0, wd1)
        pltpu.async_copy(tab_s.at[iv_s.at[0]], bs0, gs0)
        pltpu.async_copy(tab_d.at[iv_d.at[0]], bd0, gd0)

        def body(jj, carry):
            for t in range(2):
                j = jj * 2 + t
                u = 1 - t
                jn = j + 1

                @pl.when(jn < _NC)
                def _():
                    @pl.when(jn >= 2)
                    def _():
                        pltpu.make_async_copy(sb[u], out_s.at[pl.ds(0, _CH)], ws[u]).wait()
                        pltpu.make_async_copy(db[u], out_d.at[pl.ds(0, _CH)], wd[u]).wait()
                    pltpu.async_copy(tab_s.at[iv_s.at[jn]], sb[u], gs[u])
                    pltpu.async_copy(tab_d.at[iv_d.at[jn]], db[u], gd[u])

                off = pl.multiple_of(base + j * _CH, _CH)
                pltpu.make_async_copy(tab_s.at[iv_s.at[0]], sb[t], gs[t]).wait()
                pltpu.make_async_copy(tab_d.at[iv_d.at[0]], db[t], gd[t]).wait()
                pltpu.async_copy(sb[t], out_s.at[pl.ds(off, _CH)], ws[t])
                pltpu.async_copy(db[t], out_d.at[pl.ds(off, _CH)], wd[t])
            return carry

        lax.fori_loop(0, _NC // 2, body, 0)
        for t in range(2):
            pltpu.make_async_copy(sb[t], out_s.at[pl.ds(0, _CH)], ws[t]).wait()
            pltpu.make_async_copy(db[t], out_d.at[pl.ds(0, _CH)], wd[t]).wait()

    return gather


def _sc_scatter():
    mesh = plsc.VectorSubcoreMesh(core_axis_name="c", subcore_axis_name="s")

    @functools.partial(
        pl.kernel,
        mesh=mesh,
        out_type=jax.ShapeDtypeStruct((2 * _NP, 128), _F32),
        scratch_types=[
            pltpu.VMEM((_CH,), jnp.int32),
            pltpu.VMEM((_CH,), jnp.int32),
            pltpu.VMEM((_CH, 128), _F32),
            pltpu.VMEM((_CH, 128), _F32),
            pltpu.VMEM_SHARED((_NP, 128), _F32),
        ] + [pltpu.SemaphoreType.DMA] * 6,
    )
    def scat(r1_hbm, dsti_hbm, z128_hbm, aggh_out,
             iv0, iv1, rb0, rb1, acc, li0, li1, lr0, lr1, a0, a1):
        cid = lax.axis_index("c")
        sid = lax.axis_index("s")
        wid = sid * 2 + cid
        base = wid * _PERW
        pltpu.sync_copy(z128_hbm, rb0)
        for k in range(_NPS // 64):
            ns = pl.ds(sid * _NPS + k * 64, 64)
            pltpu.sync_copy(rb0, acc.at[ns])
        plsc.subcore_barrier()
        iv, rb = (iv0, iv1), (rb0, rb1)
        li, lr, ad = (li0, li1), (lr0, lr1), (a0, a1)
        pltpu.async_copy(dsti_hbm.at[wid, 0], iv0, li0)
        pltpu.async_copy(r1_hbm.at[pl.ds(base, _CH)], rb0, lr0)

        def body(jj, carry):
            for t in range(2):
                j = jj * 2 + t
                u = 1 - t
                jn = j + 1

                @pl.when(jn < _NC)
                def _():
                    @pl.when(jn >= 2)
                    def _():
                        pltpu.make_async_copy(rb[u], acc.at[iv[u]], ad[u]).wait()
                    offn = pl.multiple_of(base + jn * _CH, _CH)
                    pltpu.async_copy(dsti_hbm.at[wid, jn], iv[u], li[u])
                    pltpu.async_copy(r1_hbm.at[pl.ds(offn, _CH)], rb[u], lr[u])

                pltpu.make_async_copy(dsti_hbm.at[wid, 0], iv[t], li[t]).wait()
                pltpu.make_async_copy(r1_hbm.at[pl.ds(base, _CH)], rb[t], lr[t]).wait()
                pltpu.async_copy(rb[t], acc.at[iv[t]], ad[t], add=True)
            return carry

        lax.fori_loop(0, _NC // 2, body, 0)
        for t in range(2):
            pltpu.make_async_copy(rb[t], acc.at[iv[t]], ad[t]).wait()
        plsc.subcore_barrier()
        for k in range(_NPS // 64):
            ns = pl.ds(sid * _NPS + k * 64, 64)
            oo = pl.ds(cid * _NP + sid * _NPS + k * 64, 64)
            pltpu.sync_copy(acc.at[ns], rb0)
            pltpu.sync_copy(rb0, aggh_out.at[oo])

    return scat


def _sc_degree():
    mesh = plsc.VectorSubcoreMesh(core_axis_name="c", subcore_axis_name="s")

    @functools.partial(
        pl.kernel,
        mesh=mesh,
        out_type=jax.ShapeDtypeStruct((2 * _NP, 128), _F32),
        scratch_types=[
            pltpu.VMEM((_CH,), jnp.int32),
            pltpu.VMEM((_CH,), jnp.int32),
            pltpu.VMEM((_CH, 128), _F32),
            pltpu.VMEM_SHARED((_NP, 128), _F32),
        ] + [pltpu.SemaphoreType.DMA] * 4,
    )
    def degk(dsti_hbm, z128_hbm, ones_hbm, deg_out,
             iv0, iv1, buf, dacc, li0, li1, a0, a1):
        cid = lax.axis_index("c")
        sid = lax.axis_index("s")
        wid = sid * 2 + cid
        pltpu.sync_copy(z128_hbm, buf)
        for k in range(_NPS // 64):
            ns = pl.ds(sid * _NPS + k * 64, 64)
            pltpu.sync_copy(buf, dacc.at[ns])
        pltpu.sync_copy(ones_hbm, buf)
        plsc.subcore_barrier()
        iv, li, ad = (iv0, iv1), (li0, li1), (a0, a1)
        pltpu.async_copy(dsti_hbm.at[wid, 0], iv0, li0)

        def body(jj, carry):
            for t in range(2):
                j = jj * 2 + t
                u = 1 - t
                jn = j + 1

                @pl.when(jn < _NC)
                def _():
                    @pl.when(jn >= 2)
                    def _():
                        pltpu.make_async_copy(buf, dacc.at[iv[u]], ad[u]).wait()
                    pltpu.async_copy(dsti_hbm.at[wid, jn], iv[u], li[u])

                pltpu.make_async_copy(dsti_hbm.at[wid, 0], iv[t], li[t]).wait()
                pltpu.async_copy(buf, dacc.at[iv[t]], ad[t], add=True)
            return carry

        lax.fori_loop(0, _NC // 2, body, 0)
        for t in range(2):
            pltpu.make_async_copy(buf, dacc.at[iv[t]], ad[t]).wait()
        plsc.subcore_barrier()
        for k in range(_NPS // 64):
            ns = pl.ds(sid * _NPS + k * 64, 64)
            oo = pl.ds(cid * _NP + sid * _NPS + k * 64, 64)
            pltpu.sync_copy(dacc.at[ns], buf)
            pltpu.sync_copy(buf, deg_out.at[oo])

    return degk


def _full(r, c):
    return pl.BlockSpec((r, c), lambda i: (0, 0))


def _rows(b, c):
    return pl.BlockSpec((b, c), lambda i: (i, 0))


_SEL = np.zeros((32, 8), dtype=np.float32)
for _p in range(8):
    for _j in range(3):
        _SEL[3 * _p + _j, _p] = 1.0


def kernel(node_features, rigid_rots, rigid_trans, edge_features, res_mask,
           noising_mask, W_pts, b_pts, W_m1, b_m1, W_m2, b_m2, W_n1, b_n1,
           W_n2, b_n2, g_s, b_s, W_e1, b_e1, W_e2, b_e2, g_z, b_z, W_bb,
           b_bb, edge_index):
    pad_idx = _N + (jnp.arange(_EP - _E, dtype=jnp.int32) % (_NP - _N))
    src_p = jnp.concatenate([edge_index[0], pad_idx])
    dst_p = jnp.concatenate([edge_index[1], pad_idx])
    nf = jnp.pad(node_features, ((0, _NP - _N), (0, 0)))
    ef = jnp.pad(edge_features, ((0, _EP - _E), (0, 0)))
    rotsT = jnp.pad(rigid_rots.reshape(_N, 9), ((0, _NP - _N), (0, 7))).T
    trT = jnp.pad(rigid_trans, ((0, _NP - _N), (0, 5))).T
    rm = jnp.pad(res_mask, (0, _NP - _N)).reshape(_NP, 1)
    nm = jnp.pad(noising_mask, (0, _NP - _N)).reshape(_NP, 1)
    wptsP = jnp.pad(W_pts, ((0, 0), (0, 8)))
    bptsP = jnp.pad(b_pts, (0, 8)).reshape(1, 32)
    wm1a, wm1b, wef1, wd1 = W_m1[:256], W_m1[256:512], W_m1[512:640], W_m1[640:648]
    we1e, we1s, we1d = W_e1[:128], W_e1[128:384], W_e1[384:640]
    wn1a, wn1b = W_n1[:256], W_n1[256:512]
    wbbP = jnp.pad(W_bb, ((0, 0), (0, 2)))
    bbbP = jnp.pad(b_bb, (0, 2)).reshape(1, 8)
    sel = jnp.asarray(_SEL)
    r2 = lambda v: v.reshape(1, -1)

    a_tab, b_tab, pts = pl.pallas_call(
        _nodeprep_body,
        grid=(_NP // 1024,),
        in_specs=[_rows(1024, 256), _full(256, 32), _full(1, 32),
                  _full(256, 128), _full(256, 128)],
        out_specs=[_rows(1024, 128), _rows(1024, 128), _rows(1024, 32)],
        out_shape=[jax.ShapeDtypeStruct((_NP, 128), _F32),
                   jax.ShapeDtypeStruct((_NP, 128), _F32),
                   jax.ShapeDtypeStruct((_NP, 32), _F32)],
    )(nf, wptsP, bptsP, wm1a, wm1b)

    gpT = pl.pallas_call(
        _gpt_body,
        grid=(_NP // 2048,),
        in_specs=[_rows_t(32, 2048), _rows_t(16, 2048), _rows_t(8, 2048)],
        out_specs=_rows_t(32, 2048),
        out_shape=jax.ShapeDtypeStruct((32, _NP), _F32),
    )(pts.T, rotsT, trT)
    gp_nodes = gpT.T
    padc = jnp.zeros((_NP, 96), _F32)
    t_src = jnp.concatenate([a_tab, gp_nodes, padc], axis=1)
    t_dst = jnp.concatenate([b_tab, gp_nodes, padc], axis=1)

    src3 = src_p.reshape(_NW, _NC, _CH)
    dst3 = dst_p.reshape(_NW, _NC, _CH)
    gs1, gd1 = _sc_gather_pair(256)(t_src, t_dst, src3, dst3)

    r1 = pl.pallas_call(
        _msg_body,
        grid=(_EP // 2048,),
        in_specs=[_rows(2048, 256), _rows(2048, 256), _rows(2048, 128),
                  _full(32, 8), _full(8, 128), _full(128, 128), _full(1, 128)],
        out_specs=_rows(2048, 128),
        out_shape=jax.ShapeDtypeStruct((_EP, 128), _F32),
    )(gs1, gd1, ef, sel, wd1, wef1, r2(b_m1))

    z64 = jnp.zeros((64, 128), _F32)
    agg2 = _sc_scatter()(r1, dst3, z64)
    degf = _sc_degree()(dst3, z64, jnp.ones((64, 128), _F32))
    aggh = agg2[:_NP] + agg2[_NP:]
    deg16 = (degf[:_NP] + degf[_NP:])[:, :16]

    s, as_tab, bs_tab, upd = pl.pallas_call(
        _node_body,
        grid=(_NP // 1024,),
        in_specs=[_rows(1024, 256), _rows(1024, 128), _rows(1024, 16),
                  _full(128, 256), _full(1, 256), _full(256, 128), _full(256, 128),
                  _full(1, 128), _full(128, 256), _full(1, 256), _full(1, 256),
                  _full(1, 256), _rows(1024, 1), _rows(1024, 1),
                  _full(256, 128), _full(256, 128), _full(256, 8), _full(1, 8)],
        out_specs=[_rows(1024, 256), _rows(1024, 128), _rows(1024, 128),
                   _rows(1024, 8)],
        out_shape=[jax.ShapeDtypeStruct((_NP, 256), _F32),
                   jax.ShapeDtypeStruct((_NP, 128), _F32),
                   jax.ShapeDtypeStruct((_NP, 128), _F32),
                   jax.ShapeDtypeStruct((_NP, 8), _F32)],
    )(nf, aggh, deg16, W_m2, r2(b_m2), wn1a, wn1b, r2(b_n1), W_n2, r2(b_n2),
      r2(g_s), r2(b_s), rm, nm, we1s, we1d, wbbP, bbbP)

    nrotT, ntrT = pl.pallas_call(
        _bb_body,
        grid=(_NP // 2048,),
        in_specs=[_rows_t(8, 2048), _rows_t(16, 2048), _rows_t(8, 2048)],
        out_specs=[_rows_t(16, 2048), _rows_t(8, 2048)],
        out_shape=[jax.ShapeDtypeStruct((16, _NP), _F32),
                   jax.ShapeDtypeStruct((8, _NP), _F32)],
    )(upd.T, rotsT, trT)

    gs2, gd2 = _sc_gather_pair(128)(as_tab, bs_tab, src3, dst3)

    z = pl.pallas_call(
        _edge_body,
        grid=(_EP // 2048,),
        in_specs=[_rows(2048, 128), _rows(2048, 128), _rows(2048, 128),
                  _full(128, 128), _full(1, 128), _full(128, 128), _full(1, 128),
                  _full(1, 128), _full(1, 128)],
        out_specs=_rows(2048, 128),
        out_shape=jax.ShapeDtypeStruct((_EP, 128), _F32),
    )(ef, gs2, gd2, we1e, r2(b_e1), W_e2, r2(b_e2), r2(g_z), r2(b_z))

    s_out = s[:_N]
    new_rots = nrotT.T[:_N, :9].reshape(_N, 3, 3)
    new_trans = ntrT.T[:_N, :3]
    return (s_out, new_rots, new_trans, z[:_E])


def _rows_t(r, b):
    return pl.BlockSpec((r, b), lambda i: (0, i))
